# baseline (speedup 1.0000x reference)
import functools

import jax
import jax.numpy as jnp
from jax import lax
from jax.experimental import pallas as pl
from jax.experimental.pallas import tpu as pltpu
from jax.experimental.pallas import tpu_sc as plsc

D_MODEL = 128
NUM_CORES = 2
NUM_SUBCORES = 16
NUM_WORKERS = NUM_CORES * NUM_SUBCORES
CHUNK = 256


def _make_gather(batch: int, vocab: int):
    assert batch % (8 * NUM_WORKERS) == 0
    b_per_w = batch // NUM_WORKERS
    assert b_per_w % CHUNK == 0
    n_chunks = b_per_w // CHUNK
    assert n_chunks % 2 == 0 and n_chunks >= 4

    mesh = plsc.VectorSubcoreMesh(core_axis_name="c", subcore_axis_name="s")

    @functools.partial(
        pl.kernel,
        mesh=mesh,
        out_type=jax.ShapeDtypeStruct((batch, D_MODEL), jnp.float32),
        scratch_types=[
            pltpu.VMEM((CHUNK,), jnp.int32),
            pltpu.VMEM((CHUNK,), jnp.int32),
            pltpu.VMEM((2, CHUNK, D_MODEL), jnp.float32),
            pltpu.VMEM_SHARED((vocab, D_MODEL), jnp.float32),
            pltpu.SemaphoreType.DMA,
            pltpu.SemaphoreType.DMA,
            pltpu.SemaphoreType.DMA,
            pltpu.SemaphoreType.DMA,
            pltpu.SemaphoreType.DMA,
            pltpu.SemaphoreType.DMA,
        ],
    )
    def gather_kernel(idx_hbm, table_hbm, out_hbm, ix0, ix1, rows_v,
                      table_sp, si0, si1, sg0, sg1, so0, so1):
        wid = lax.axis_index("s") * NUM_CORES + lax.axis_index("c")
        base = wid * b_per_w
        ix = (ix0, ix1)
        si = (si0, si1)
        sg = (sg0, sg1)
        so = (so0, so1)

        @pl.when(lax.axis_index("s") == 0)
        def _stage_table():
            pltpu.sync_copy(table_hbm, table_sp)

        plsc.subcore_barrier()

        def idxc(c, p):
            return pltpu.make_async_copy(
                idx_hbm.at[pl.ds(base + c * CHUNK, CHUNK)], ix[p], si[p])

        def gat(p):
            return pltpu.make_async_copy(
                table_sp.at[ix[p]], rows_v.at[p], sg[p])

        def outc(c, p):
            return pltpu.make_async_copy(
                rows_v.at[p], out_hbm.at[pl.ds(base + c * CHUNK, CHUNK)],
                so[p])

        idxc(0, 0).start()
        idxc(1, 1).start()
        idxc(0, 0).wait()
        gat(0).start()
        idxc(1, 1).wait()
        gat(0).wait()
        gat(1).start()
        idxc(2, 0).start()
        outc(0, 0).start()

        def step(c, p, q):
            gat(p).wait()
            idxc(c + 1, q).wait()
            outc(c - 1, q).wait()
            gat(q).start()
            idxc(c + 2, p).start()
            outc(c, p).start()

        def pair_body(i2, carry):
            c = 2 * i2 + 1
            step(c, 1, 0)
            step(c + 1, 0, 1)
            return carry

        lax.fori_loop(0, (n_chunks - 4) // 2, pair_body, 0)

        n = n_chunks
        gat(1).wait()
        idxc(n - 2, 0).wait()
        outc(n - 4, 0).wait()
        gat(0).start()
        idxc(n - 1, 1).start()
        outc(n - 3, 1).start()

        gat(0).wait()
        idxc(n - 1, 1).wait()
        outc(n - 3, 1).wait()
        gat(1).start()
        outc(n - 2, 0).start()

        gat(1).wait()
        outc(n - 2, 0).wait()
        outc(n - 1, 1).start()
        outc(n - 1, 1).wait()

    return gather_kernel


def kernel(x, table):
    batch, hist = x.shape
    idx = x.reshape(-1)
    out = _make_gather(batch * hist, table.shape[0])(idx, table)
    return out.reshape(batch, hist, D_MODEL)


# speedup vs baseline: 1.0013x; 1.0013x over previous
import functools

import jax
import jax.numpy as jnp
from jax import lax
from jax.experimental import pallas as pl
from jax.experimental.pallas import tpu as pltpu
from jax.experimental.pallas import tpu_sc as plsc

D_MODEL = 128
NUM_CORES = 2
NUM_SUBCORES = 16
NUM_WORKERS = NUM_CORES * NUM_SUBCORES
CHUNK = 320


def _make_gather(batch: int, vocab: int):
    assert batch % (8 * NUM_WORKERS) == 0
    b_per_w = batch // NUM_WORKERS
    assert b_per_w % CHUNK == 0
    n_chunks = b_per_w // CHUNK
    assert n_chunks % 2 == 0 and n_chunks >= 4

    mesh = plsc.VectorSubcoreMesh(core_axis_name="c", subcore_axis_name="s")

    @functools.partial(
        pl.kernel,
        mesh=mesh,
        out_type=jax.ShapeDtypeStruct((batch, D_MODEL), jnp.float32),
        scratch_types=[
            pltpu.VMEM((CHUNK,), jnp.int32),
            pltpu.VMEM((CHUNK,), jnp.int32),
            pltpu.VMEM((2, CHUNK, D_MODEL), jnp.float32),
            pltpu.VMEM_SHARED((vocab, D_MODEL), jnp.float32),
            pltpu.SemaphoreType.DMA,
            pltpu.SemaphoreType.DMA,
            pltpu.SemaphoreType.DMA,
            pltpu.SemaphoreType.DMA,
            pltpu.SemaphoreType.DMA,
            pltpu.SemaphoreType.DMA,
        ],
    )
    def gather_kernel(idx_hbm, table_hbm, out_hbm, ix0, ix1, rows_v,
                      table_sp, si0, si1, sg0, sg1, so0, so1):
        wid = lax.axis_index("s") * NUM_CORES + lax.axis_index("c")
        base = wid * b_per_w
        ix = (ix0, ix1)
        si = (si0, si1)
        sg = (sg0, sg1)
        so = (so0, so1)

        @pl.when(lax.axis_index("s") == 0)
        def _stage_table():
            pltpu.sync_copy(table_hbm, table_sp)

        plsc.subcore_barrier()

        def idxc(c, p):
            return pltpu.make_async_copy(
                idx_hbm.at[pl.ds(base + c * CHUNK, CHUNK)], ix[p], si[p])

        def gat(p):
            return pltpu.make_async_copy(
                table_sp.at[ix[p]], rows_v.at[p], sg[p])

        def outc(c, p):
            return pltpu.make_async_copy(
                rows_v.at[p], out_hbm.at[pl.ds(base + c * CHUNK, CHUNK)],
                so[p])

        idxc(0, 0).start()
        idxc(1, 1).start()
        idxc(0, 0).wait()
        gat(0).start()
        idxc(1, 1).wait()
        gat(0).wait()
        gat(1).start()
        idxc(2, 0).start()
        outc(0, 0).start()

        def step(c, p, q):
            gat(p).wait()
            idxc(c + 1, q).wait()
            outc(c - 1, q).wait()
            gat(q).start()
            idxc(c + 2, p).start()
            outc(c, p).start()

        def pair_body(i2, carry):
            c = 2 * i2 + 1
            step(c, 1, 0)
            step(c + 1, 0, 1)
            return carry

        lax.fori_loop(0, (n_chunks - 4) // 2, pair_body, 0)

        n = n_chunks
        gat(1).wait()
        idxc(n - 2, 0).wait()
        outc(n - 4, 0).wait()
        gat(0).start()
        idxc(n - 1, 1).start()
        outc(n - 3, 1).start()

        gat(0).wait()
        idxc(n - 1, 1).wait()
        outc(n - 3, 1).wait()
        gat(1).start()
        outc(n - 2, 0).start()

        gat(1).wait()
        outc(n - 2, 0).wait()
        outc(n - 1, 1).start()
        outc(n - 1, 1).wait()

    return gather_kernel


def kernel(x, table):
    batch, hist = x.shape
    idx = x.reshape(-1)
    out = _make_gather(batch * hist, table.shape[0])(idx, table)
    return out.reshape(batch, hist, D_MODEL)
